# baseline (speedup 1.0000x reference)
import functools

import jax
import jax.numpy as jnp
from jax.experimental import pallas as pl
from jax.experimental.pallas import tpu as pltpu
from jax.experimental.pallas import tpu_sc as plsc

K_NB = 64
_NC, _NS = 2, 16
_NW = _NC * _NS


def _sc_gather(table, idx, CH):
    G = idx.shape[0]
    b_per_w = G // _NW
    n_chunks = b_per_w // CH
    mesh = plsc.VectorSubcoreMesh(core_axis_name="c", subcore_axis_name="s")

    @functools.partial(
        pl.kernel, mesh=mesh,
        out_type=jax.ShapeDtypeStruct((G, 128), jnp.float32),
        scratch_types=[
            pltpu.VMEM((CH,), jnp.int32),
            pltpu.VMEM((CH, 128), jnp.float32),
            pltpu.SemaphoreType.DMA,
        ],
    )
    def k(table_hbm, idx_hbm, out_hbm, idx_v, rows_v, sem):
        wid = jax.lax.axis_index("s") * _NC + jax.lax.axis_index("c")
        base = wid * b_per_w

        @pl.loop(0, n_chunks)
        def _chunk(c):
            off = base + c * CH
            pltpu.sync_copy(idx_hbm.at[pl.ds(off, CH)], idx_v)
            pltpu.async_copy(table_hbm.at[idx_v], rows_v, sem).wait()
            pltpu.sync_copy(rows_v, out_hbm.at[pl.ds(off, CH)])

    return k(table, idx)


def _fps_body(pos_ref, cen_ref, *, n_samples, SM, LM):
    px = pos_ref[0, 0]
    py = pos_ref[0, 1]
    pz = pos_ref[0, 2]
    S, L = px.shape
    jj = (jax.lax.broadcasted_iota(jnp.int32, (S, L), 0) * L
          + jax.lax.broadcasted_iota(jnp.int32, (S, L), 1))
    ii = (jax.lax.broadcasted_iota(jnp.int32, (SM, LM), 0) * LM
          + jax.lax.broadcasted_iota(jnp.int32, (SM, LM), 1))
    big = jnp.int32(S * L)

    c0x = px[0, 0]
    c0y = py[0, 0]
    c0z = pz[0, 0]
    dx = px - c0x
    dy = py - c0y
    dz = pz - c0z
    dists = (dx * dx + dy * dy) + dz * dz

    cxa = jnp.where(ii == 0, c0x, 0.0)
    cya = jnp.where(ii == 0, c0y, 0.0)
    cza = jnp.where(ii == 0, c0z, 0.0)

    def body(i, state):
        dists, cxa, cya, cza = state
        m = jnp.max(dists)
        nxt = jnp.min(jnp.where(dists == m, jj, big))
        sel = (jj == nxt).astype(jnp.float32)
        cx = jnp.sum(px * sel)
        cy = jnp.sum(py * sel)
        cz = jnp.sum(pz * sel)
        dx = px - cx
        dy = py - cy
        dz = pz - cz
        d = (dx * dx + dy * dy) + dz * dz
        dists = jnp.minimum(dists, d)
        hit = ii == i
        cxa = jnp.where(hit, cx, cxa)
        cya = jnp.where(hit, cy, cya)
        cza = jnp.where(hit, cz, cza)
        return dists, cxa, cya, cza

    dists, cxa, cya, cza = jax.lax.fori_loop(
        1, n_samples, body, (dists, cxa, cya, cza))
    cen_ref[0, 0] = cxa
    cen_ref[0, 1] = cya
    cen_ref[0, 2] = cza


def _fps(pos, n_samples):
    B, N, _ = pos.shape
    S, L = 8, N // 8
    SM, LM = 8, n_samples // 8
    pos_t = pos.transpose(0, 2, 1).reshape(B, 3, S, L)
    cen = pl.pallas_call(
        functools.partial(_fps_body, n_samples=n_samples, SM=SM, LM=LM),
        grid=(B,),
        in_specs=[pl.BlockSpec((1, 3, S, L), lambda b: (b, 0, 0, 0))],
        out_specs=pl.BlockSpec((1, 3, SM, LM), lambda b: (b, 0, 0, 0)),
        out_shape=jax.ShapeDtypeStruct((B, 3, SM, LM), jnp.float32),
        compiler_params=pltpu.CompilerParams(
            dimension_semantics=("arbitrary",)),
        interpret=False,
    )(pos_t)
    return cen.reshape(B, 3, n_samples).transpose(0, 2, 1)


def _scores_body(cen_ref, pos_ref, out_ref, *, r2):
    cen = cen_ref[0]
    pos_t = pos_ref[0]
    cn = jnp.sum(cen * cen, axis=1, keepdims=True)
    pn = jnp.sum(pos_t * pos_t, axis=0, keepdims=True)
    cp = jnp.dot(cen, pos_t, preferred_element_type=jnp.float32)
    d2 = (cn + pn) - 2.0 * cp
    out_ref[0] = jnp.where(d2 <= r2, -d2, -jnp.inf)


def _scores(centers, pos, r):
    B, M, _ = centers.shape
    N = pos.shape[1]
    MB = 256
    pos_t = pos.transpose(0, 2, 1)
    return pl.pallas_call(
        functools.partial(_scores_body, r2=r * r),
        grid=(B, M // MB),
        in_specs=[
            pl.BlockSpec((1, MB, 3), lambda b, m: (b, m, 0)),
            pl.BlockSpec((1, 3, N), lambda b, m: (b, 0, 0)),
        ],
        out_specs=pl.BlockSpec((1, MB, N), lambda b, m: (b, m, 0)),
        out_shape=jax.ShapeDtypeStruct((B, M, N), jnp.float32),
        compiler_params=pltpu.CompilerParams(
            dimension_semantics=("parallel", "arbitrary")),
        interpret=False,
    )(centers, pos_t)


def _mlp_body(rows_ref, crep_ref, v_ref, w1f_ref, w1r_ref, b1_ref,
              w2_ref, b2_ref, w3_ref, b3_ref, out_ref, *, MB, Cf):
    rows = rows_ref[0]
    x_j = rows[:, 3:3 + Cf]
    rel = rows[:, 0:3] - crep_ref[0]
    h = jnp.maximum(jnp.dot(x_j, w1f_ref[...],
                            preferred_element_type=jnp.float32)
                    + jnp.dot(rel, w1r_ref[...],
                              preferred_element_type=jnp.float32)
                    + b1_ref[...], 0.0)
    h = jnp.maximum(jnp.dot(h, w2_ref[...],
                            preferred_element_type=jnp.float32)
                    + b2_ref[...], 0.0)
    h = jnp.maximum(jnp.dot(h, w3_ref[...],
                            preferred_element_type=jnp.float32)
                    + b3_ref[...], 0.0)
    C = h.shape[-1]
    v = v_ref[0]
    h = jnp.where(v > 0.0, h, -jnp.inf)
    h = h.reshape(MB, K_NB, C)
    out = jnp.max(h, axis=1)
    out_ref[0] = jnp.where(jnp.isfinite(out), out, 0.0)


def _mlp_pool(rows, crep, valid, Cf, params):
    B, MK, _ = rows.shape
    M = MK // K_NB
    MB = min(M, 128)
    (W1, b1), (W2, b2), (W3, b3) = params
    W1f, W1r = W1[:Cf], W1[Cf:]
    Cout = W3.shape[1]
    vf = valid.astype(jnp.float32).reshape(B, MK, 1)
    wspec = lambda w: pl.BlockSpec(w.shape, lambda b, m: (0,) * w.ndim)
    b1r, b2r, b3r = (b.reshape(1, -1) for b in (b1, b2, b3))
    return pl.pallas_call(
        functools.partial(_mlp_body, MB=MB, Cf=Cf),
        grid=(B, M // MB),
        in_specs=[
            pl.BlockSpec((1, MB * K_NB, 128), lambda b, m: (b, m, 0)),
            pl.BlockSpec((1, MB * K_NB, 3), lambda b, m: (b, m, 0)),
            pl.BlockSpec((1, MB * K_NB, 1), lambda b, m: (b, m, 0)),
            wspec(W1f), wspec(W1r), wspec(b1r),
            wspec(W2), wspec(b2r), wspec(W3), wspec(b3r),
        ],
        out_specs=pl.BlockSpec((1, MB, Cout), lambda b, m: (b, m, 0)),
        out_shape=jax.ShapeDtypeStruct((B, M, Cout), jnp.float32),
        compiler_params=pltpu.CompilerParams(
            dimension_semantics=("parallel", "arbitrary")),
        interpret=False,
    )(rows, crep, vf, W1f, W1r, b1r, W2, b2r, W3, b3r)


def _set_conv(feat, pos, r, M, params):
    B, N, _ = pos.shape
    centers = _fps(pos, M)
    scores = _scores(centers, pos, r)
    NCH = N // 512
    s4 = scores.reshape(B, M, NCH, 512)
    v1, i1 = jax.lax.top_k(s4, K_NB)
    i1 = i1 + (jnp.arange(NCH, dtype=jnp.int32) * 512)[None, None, :, None]
    v1 = v1.reshape(B, M, NCH * K_NB)
    i1 = i1.reshape(B, M, NCH * K_NB)
    vals, i2 = jax.lax.top_k(v1, K_NB)
    nbr = jnp.take_along_axis(i1, i2, axis=2)
    valid = vals > -jnp.inf
    Cf = feat.shape[-1]
    table = jnp.concatenate([pos, feat], axis=-1).reshape(B * N, 3 + Cf)
    table = jnp.pad(table, ((0, 0), (0, 128 - (3 + Cf))))
    flat_idx = (nbr + (jnp.arange(B, dtype=jnp.int32) * N)[:, None, None])
    rows = _sc_gather(table, flat_idx.reshape(-1), 512)
    rows = rows.reshape(B, M * K_NB, 128)
    crep = jnp.broadcast_to(centers[:, :, None, :],
                            (B, M, K_NB, 3)).reshape(B, M * K_NB, 3)
    out = _mlp_pool(rows, crep, valid, Cf, params)
    return out, centers


def kernel(x, W1_1, b1_1, W1_2, b1_2, W1_3, b1_3,
           W2_1, b2_1, W2_2, b2_2, W2_3, b2_3):
    B, N, _ = x.shape
    feat = x[:, :, 3:]
    pos = x[:, :, :3]
    params1 = [(W1_1, b1_1), (W1_2, b1_2), (W1_3, b1_3)]
    params2 = [(W2_1, b2_1), (W2_2, b2_2), (W2_3, b2_3)]
    f1, p1 = _set_conv(feat, pos, 0.5, N // 2, params1)
    f2, p2 = _set_conv(f1, p1, 1.0, N // 8, params2)
    M2 = f2.shape[1]
    batch = jnp.repeat(jnp.arange(B, dtype=jnp.int32), M2)
    return (f2.reshape(B * M2, -1), p2.reshape(B * M2, 3), batch)


# speedup vs baseline: 2.9786x; 2.9786x over previous
import functools

import jax
import jax.numpy as jnp
from jax.experimental import pallas as pl
from jax.experimental.pallas import tpu as pltpu
from jax.experimental.pallas import tpu_sc as plsc

K_NB = 64
_NC, _NS = 2, 16
_NW = _NC * _NS


def _sc_gather(table, idx, CH):
    G = idx.shape[0]
    b_per_w = G // _NW
    n_chunks = b_per_w // CH
    mesh = plsc.VectorSubcoreMesh(core_axis_name="c", subcore_axis_name="s")

    @functools.partial(
        pl.kernel, mesh=mesh,
        out_type=jax.ShapeDtypeStruct((G, 128), jnp.float32),
        scratch_types=[
            pltpu.VMEM((CH,), jnp.int32),
            pltpu.VMEM((CH, 128), jnp.float32),
            pltpu.SemaphoreType.DMA,
        ],
    )
    def k(table_hbm, idx_hbm, out_hbm, idx_v, rows_v, sem):
        wid = jax.lax.axis_index("s") * _NC + jax.lax.axis_index("c")
        base = wid * b_per_w

        @pl.loop(0, n_chunks)
        def _chunk(c):
            off = base + c * CH
            pltpu.sync_copy(idx_hbm.at[pl.ds(off, CH)], idx_v)
            pltpu.async_copy(table_hbm.at[idx_v], rows_v, sem).wait()
            pltpu.sync_copy(rows_v, out_hbm.at[pl.ds(off, CH)])

    return k(table, idx)


def _fps_body(pos_ref, cen_ref, *, n_samples, SM, LM):
    px = pos_ref[0, 0]
    py = pos_ref[0, 1]
    pz = pos_ref[0, 2]
    S, L = px.shape
    jj = (jax.lax.broadcasted_iota(jnp.int32, (S, L), 0) * L
          + jax.lax.broadcasted_iota(jnp.int32, (S, L), 1))
    ii = (jax.lax.broadcasted_iota(jnp.int32, (SM, LM), 0) * LM
          + jax.lax.broadcasted_iota(jnp.int32, (SM, LM), 1))
    big = jnp.int32(S * L)

    c0x = px[0, 0]
    c0y = py[0, 0]
    c0z = pz[0, 0]
    dx = px - c0x
    dy = py - c0y
    dz = pz - c0z
    dists = (dx * dx + dy * dy) + dz * dz

    cxa = jnp.where(ii == 0, c0x, 0.0)
    cya = jnp.where(ii == 0, c0y, 0.0)
    cza = jnp.where(ii == 0, c0z, 0.0)

    def body(i, state):
        dists, cxa, cya, cza = state
        m = jnp.max(dists)
        nxt = jnp.min(jnp.where(dists == m, jj, big))
        sel = (jj == nxt).astype(jnp.float32)
        cx = jnp.sum(px * sel)
        cy = jnp.sum(py * sel)
        cz = jnp.sum(pz * sel)
        dx = px - cx
        dy = py - cy
        dz = pz - cz
        d = (dx * dx + dy * dy) + dz * dz
        dists = jnp.minimum(dists, d)
        hit = ii == i
        cxa = jnp.where(hit, cx, cxa)
        cya = jnp.where(hit, cy, cya)
        cza = jnp.where(hit, cz, cza)
        return dists, cxa, cya, cza

    dists, cxa, cya, cza = jax.lax.fori_loop(
        1, n_samples, body, (dists, cxa, cya, cza))
    cen_ref[0, 0] = cxa
    cen_ref[0, 1] = cya
    cen_ref[0, 2] = cza


def _fps(pos, n_samples):
    B, N, _ = pos.shape
    S, L = 8, N // 8
    SM, LM = 8, n_samples // 8
    pos_t = pos.transpose(0, 2, 1).reshape(B, 3, S, L)
    cen = pl.pallas_call(
        functools.partial(_fps_body, n_samples=n_samples, SM=SM, LM=LM),
        grid=(B,),
        in_specs=[pl.BlockSpec((1, 3, S, L), lambda b: (b, 0, 0, 0))],
        out_specs=pl.BlockSpec((1, 3, SM, LM), lambda b: (b, 0, 0, 0)),
        out_shape=jax.ShapeDtypeStruct((B, 3, SM, LM), jnp.float32),
        compiler_params=pltpu.CompilerParams(
            dimension_semantics=("parallel",)),
        interpret=False,
    )(pos_t)
    return cen.reshape(B, 3, n_samples).transpose(0, 2, 1)


def _scores_body(cen_ref, pos_ref, out_ref, *, r2):
    cen = cen_ref[0]
    pos_t = pos_ref[0]
    cn = jnp.sum(cen * cen, axis=1, keepdims=True)
    pn = jnp.sum(pos_t * pos_t, axis=0, keepdims=True)
    cp = jnp.dot(cen, pos_t, preferred_element_type=jnp.float32)
    d2 = (cn + pn) - 2.0 * cp
    out_ref[0] = jnp.where(d2 <= r2, -d2, -jnp.inf)


def _scores(centers, pos, r):
    B, M, _ = centers.shape
    N = pos.shape[1]
    MB = 256
    pos_t = pos.transpose(0, 2, 1)
    return pl.pallas_call(
        functools.partial(_scores_body, r2=r * r),
        grid=(B, M // MB),
        in_specs=[
            pl.BlockSpec((1, MB, 3), lambda b, m: (b, m, 0)),
            pl.BlockSpec((1, 3, N), lambda b, m: (b, 0, 0)),
        ],
        out_specs=pl.BlockSpec((1, MB, N), lambda b, m: (b, m, 0)),
        out_shape=jax.ShapeDtypeStruct((B, M, N), jnp.float32),
        compiler_params=pltpu.CompilerParams(
            dimension_semantics=("parallel", "arbitrary")),
        interpret=False,
    )(centers, pos_t)


def _mlp_body(rows_ref, crep_ref, v_ref, w1f_ref, w1r_ref, b1_ref,
              w2_ref, b2_ref, w3_ref, b3_ref, out_ref, *, MB, Cf):
    rows = rows_ref[0]
    x_j = rows[:, 3:3 + Cf]
    rel = rows[:, 0:3] - crep_ref[0]
    h = jnp.maximum(jnp.dot(x_j, w1f_ref[...],
                            preferred_element_type=jnp.float32)
                    + jnp.dot(rel, w1r_ref[...],
                              preferred_element_type=jnp.float32)
                    + b1_ref[...], 0.0)
    h = jnp.maximum(jnp.dot(h, w2_ref[...],
                            preferred_element_type=jnp.float32)
                    + b2_ref[...], 0.0)
    h = jnp.maximum(jnp.dot(h, w3_ref[...],
                            preferred_element_type=jnp.float32)
                    + b3_ref[...], 0.0)
    C = h.shape[-1]
    v = v_ref[0]
    h = jnp.where(v > 0.0, h, -jnp.inf)
    h = h.reshape(MB, K_NB, C)
    out = jnp.max(h, axis=1)
    out_ref[0] = jnp.where(jnp.isfinite(out), out, 0.0)


def _mlp_pool(rows, crep, valid, Cf, params):
    B, MK, _ = rows.shape
    M = MK // K_NB
    MB = min(M, 128)
    (W1, b1), (W2, b2), (W3, b3) = params
    W1f, W1r = W1[:Cf], W1[Cf:]
    Cout = W3.shape[1]
    vf = valid.astype(jnp.float32).reshape(B, MK, 1)
    wspec = lambda w: pl.BlockSpec(w.shape, lambda b, m: (0,) * w.ndim)
    b1r, b2r, b3r = (b.reshape(1, -1) for b in (b1, b2, b3))
    return pl.pallas_call(
        functools.partial(_mlp_body, MB=MB, Cf=Cf),
        grid=(B, M // MB),
        in_specs=[
            pl.BlockSpec((1, MB * K_NB, 128), lambda b, m: (b, m, 0)),
            pl.BlockSpec((1, MB * K_NB, 3), lambda b, m: (b, m, 0)),
            pl.BlockSpec((1, MB * K_NB, 1), lambda b, m: (b, m, 0)),
            wspec(W1f), wspec(W1r), wspec(b1r),
            wspec(W2), wspec(b2r), wspec(W3), wspec(b3r),
        ],
        out_specs=pl.BlockSpec((1, MB, Cout), lambda b, m: (b, m, 0)),
        out_shape=jax.ShapeDtypeStruct((B, M, Cout), jnp.float32),
        compiler_params=pltpu.CompilerParams(
            dimension_semantics=("parallel", "arbitrary")),
        interpret=False,
    )(rows, crep, vf, W1f, W1r, b1r, W2, b2r, W3, b3r)


def _set_conv(feat, pos, r, M, params):
    B, N, _ = pos.shape
    centers = _fps(pos, M)
    scores = _scores(centers, pos, r)
    vals, nbr = jax.lax.top_k(scores, K_NB)
    valid = vals > -jnp.inf
    Cf = feat.shape[-1]
    table = jnp.concatenate([pos, feat], axis=-1).reshape(B * N, 3 + Cf)
    table = jnp.pad(table, ((0, 0), (0, 128 - (3 + Cf))))
    flat_idx = (nbr + (jnp.arange(B, dtype=jnp.int32) * N)[:, None, None])
    rows = _sc_gather(table, flat_idx.reshape(-1), 512)
    rows = rows.reshape(B, M * K_NB, 128)
    crep = jnp.broadcast_to(centers[:, :, None, :],
                            (B, M, K_NB, 3)).reshape(B, M * K_NB, 3)
    out = _mlp_pool(rows, crep, valid, Cf, params)
    return out, centers


def kernel(x, W1_1, b1_1, W1_2, b1_2, W1_3, b1_3,
           W2_1, b2_1, W2_2, b2_2, W2_3, b2_3):
    B, N, _ = x.shape
    feat = x[:, :, 3:]
    pos = x[:, :, :3]
    params1 = [(W1_1, b1_1), (W1_2, b1_2), (W1_3, b1_3)]
    params2 = [(W2_1, b2_1), (W2_2, b2_2), (W2_3, b2_3)]
    f1, p1 = _set_conv(feat, pos, 0.5, N // 2, params1)
    f2, p2 = _set_conv(f1, p1, 1.0, N // 8, params2)
    M2 = f2.shape[1]
    batch = jnp.repeat(jnp.arange(B, dtype=jnp.int32), M2)
    return (f2.reshape(B * M2, -1), p2.reshape(B * M2, 3), batch)


# speedup vs baseline: 3.2278x; 1.0837x over previous
import functools

import jax
import jax.numpy as jnp
from jax.experimental import pallas as pl
from jax.experimental.pallas import tpu as pltpu
from jax.experimental.pallas import tpu_sc as plsc

K_NB = 64
_NC, _NS = 2, 16
_NW = _NC * _NS


def _sc_gather(table, idx, CH):
    G = idx.shape[0]
    b_per_w = G // _NW
    n_chunks = b_per_w // CH
    mesh = plsc.VectorSubcoreMesh(core_axis_name="c", subcore_axis_name="s")

    @functools.partial(
        pl.kernel, mesh=mesh,
        out_type=jax.ShapeDtypeStruct((G, 128), jnp.float32),
        scratch_types=[
            pltpu.VMEM((CH,), jnp.int32),
            pltpu.VMEM((CH, 128), jnp.float32),
            pltpu.SemaphoreType.DMA,
        ],
    )
    def k(table_hbm, idx_hbm, out_hbm, idx_v, rows_v, sem):
        wid = jax.lax.axis_index("s") * _NC + jax.lax.axis_index("c")
        base = wid * b_per_w

        @pl.loop(0, n_chunks)
        def _chunk(c):
            off = base + c * CH
            pltpu.sync_copy(idx_hbm.at[pl.ds(off, CH)], idx_v)
            pltpu.async_copy(table_hbm.at[idx_v], rows_v, sem).wait()
            pltpu.sync_copy(rows_v, out_hbm.at[pl.ds(off, CH)])

    return k(table, idx)


def _fps_body(pos_ref, psm_ref, cen_ref, *, n_samples, SM, LM):
    px = pos_ref[0, 0]
    py = pos_ref[0, 1]
    pz = pos_ref[0, 2]
    S, L = px.shape
    jj = (jax.lax.broadcasted_iota(jnp.int32, (S, L), 0) * L
          + jax.lax.broadcasted_iota(jnp.int32, (S, L), 1))
    ii = (jax.lax.broadcasted_iota(jnp.int32, (SM, LM), 0) * LM
          + jax.lax.broadcasted_iota(jnp.int32, (SM, LM), 1))
    big = jnp.int32(S * L)

    c0x = px[0, 0]
    c0y = py[0, 0]
    c0z = pz[0, 0]
    dx = px - c0x
    dy = py - c0y
    dz = pz - c0z
    dists = (dx * dx + dy * dy) + dz * dz

    cxa = jnp.where(ii == 0, c0x, 0.0)
    cya = jnp.where(ii == 0, c0y, 0.0)
    cza = jnp.where(ii == 0, c0z, 0.0)

    def body(i, state):
        dists, cxa, cya, cza = state
        m = jnp.max(dists)
        nxt = jnp.min(jnp.where(dists == m, jj, big))
        cx = psm_ref[0, 0, nxt]
        cy = psm_ref[0, 1, nxt]
        cz = psm_ref[0, 2, nxt]
        dx = px - cx
        dy = py - cy
        dz = pz - cz
        d = (dx * dx + dy * dy) + dz * dz
        dists = jnp.minimum(dists, d)
        hit = ii == i
        cxa = jnp.where(hit, cx, cxa)
        cya = jnp.where(hit, cy, cya)
        cza = jnp.where(hit, cz, cza)
        return dists, cxa, cya, cza

    dists, cxa, cya, cza = jax.lax.fori_loop(
        1, n_samples, body, (dists, cxa, cya, cza))
    cen_ref[0, 0] = cxa
    cen_ref[0, 1] = cya
    cen_ref[0, 2] = cza


def _fps(pos, n_samples):
    B, N, _ = pos.shape
    S, L = 8, N // 8
    SM, LM = 8, n_samples // 8
    pos_t = pos.transpose(0, 2, 1).reshape(B, 3, S, L)
    cen = pl.pallas_call(
        functools.partial(_fps_body, n_samples=n_samples, SM=SM, LM=LM),
        grid=(B,),
        in_specs=[
            pl.BlockSpec((1, 3, S, L), lambda b: (b, 0, 0, 0)),
            pl.BlockSpec((1, 3, S * L), lambda b: (b, 0, 0),
                         memory_space=pltpu.SMEM),
        ],
        out_specs=pl.BlockSpec((1, 3, SM, LM), lambda b: (b, 0, 0, 0)),
        out_shape=jax.ShapeDtypeStruct((B, 3, SM, LM), jnp.float32),
        compiler_params=pltpu.CompilerParams(
            dimension_semantics=("parallel",)),
        interpret=False,
    )(pos_t, pos_t.reshape(B, 3, N))
    return cen.reshape(B, 3, n_samples).transpose(0, 2, 1)


def _scores_body(cen_ref, pos_ref, out_ref, *, r2):
    cen = cen_ref[0]
    pos_t = pos_ref[0]
    cn = jnp.sum(cen * cen, axis=1, keepdims=True)
    pn = jnp.sum(pos_t * pos_t, axis=0, keepdims=True)
    cp = jnp.dot(cen, pos_t, preferred_element_type=jnp.float32)
    d2 = (cn + pn) - 2.0 * cp
    out_ref[0] = jnp.where(d2 <= r2, -d2, -jnp.inf)


def _scores(centers, pos, r):
    B, M, _ = centers.shape
    N = pos.shape[1]
    MB = 256
    pos_t = pos.transpose(0, 2, 1)
    return pl.pallas_call(
        functools.partial(_scores_body, r2=r * r),
        grid=(B, M // MB),
        in_specs=[
            pl.BlockSpec((1, MB, 3), lambda b, m: (b, m, 0)),
            pl.BlockSpec((1, 3, N), lambda b, m: (b, 0, 0)),
        ],
        out_specs=pl.BlockSpec((1, MB, N), lambda b, m: (b, m, 0)),
        out_shape=jax.ShapeDtypeStruct((B, M, N), jnp.float32),
        compiler_params=pltpu.CompilerParams(
            dimension_semantics=("parallel", "arbitrary")),
        interpret=False,
    )(centers, pos_t)


def _mlp_body(rows_ref, crep_ref, v_ref, w1f_ref, w1r_ref, b1_ref,
              w2_ref, b2_ref, w3_ref, b3_ref, out_ref, *, MB, Cf):
    rows = rows_ref[0]
    x_j = rows[:, 3:3 + Cf]
    rel = rows[:, 0:3] - crep_ref[0]
    h = jnp.maximum(jnp.dot(x_j, w1f_ref[...],
                            preferred_element_type=jnp.float32)
                    + jnp.dot(rel, w1r_ref[...],
                              preferred_element_type=jnp.float32)
                    + b1_ref[...], 0.0)
    h = jnp.maximum(jnp.dot(h, w2_ref[...],
                            preferred_element_type=jnp.float32)
                    + b2_ref[...], 0.0)
    h = jnp.maximum(jnp.dot(h, w3_ref[...],
                            preferred_element_type=jnp.float32)
                    + b3_ref[...], 0.0)
    C = h.shape[-1]
    v = v_ref[0]
    h = jnp.where(v > 0.0, h, -jnp.inf)
    h = h.reshape(MB, K_NB, C)
    out = jnp.max(h, axis=1)
    out_ref[0] = jnp.where(jnp.isfinite(out), out, 0.0)


def _mlp_pool(rows, crep, valid, Cf, params):
    B, MK, _ = rows.shape
    M = MK // K_NB
    MB = min(M, 128)
    (W1, b1), (W2, b2), (W3, b3) = params
    W1f, W1r = W1[:Cf], W1[Cf:]
    Cout = W3.shape[1]
    vf = valid.astype(jnp.float32).reshape(B, MK, 1)
    wspec = lambda w: pl.BlockSpec(w.shape, lambda b, m: (0,) * w.ndim)
    b1r, b2r, b3r = (b.reshape(1, -1) for b in (b1, b2, b3))
    return pl.pallas_call(
        functools.partial(_mlp_body, MB=MB, Cf=Cf),
        grid=(B, M // MB),
        in_specs=[
            pl.BlockSpec((1, MB * K_NB, 128), lambda b, m: (b, m, 0)),
            pl.BlockSpec((1, MB * K_NB, 3), lambda b, m: (b, m, 0)),
            pl.BlockSpec((1, MB * K_NB, 1), lambda b, m: (b, m, 0)),
            wspec(W1f), wspec(W1r), wspec(b1r),
            wspec(W2), wspec(b2r), wspec(W3), wspec(b3r),
        ],
        out_specs=pl.BlockSpec((1, MB, Cout), lambda b, m: (b, m, 0)),
        out_shape=jax.ShapeDtypeStruct((B, M, Cout), jnp.float32),
        compiler_params=pltpu.CompilerParams(
            dimension_semantics=("parallel", "arbitrary")),
        interpret=False,
    )(rows, crep, vf, W1f, W1r, b1r, W2, b2r, W3, b3r)


def _set_conv(feat, pos, r, M, params):
    B, N, _ = pos.shape
    centers = _fps(pos, M)
    scores = _scores(centers, pos, r)
    vals, nbr = jax.lax.top_k(scores, K_NB)
    valid = vals > -jnp.inf
    Cf = feat.shape[-1]
    table = jnp.concatenate([pos, feat], axis=-1).reshape(B * N, 3 + Cf)
    table = jnp.pad(table, ((0, 0), (0, 128 - (3 + Cf))))
    flat_idx = (nbr + (jnp.arange(B, dtype=jnp.int32) * N)[:, None, None])
    rows = _sc_gather(table, flat_idx.reshape(-1), 512)
    rows = rows.reshape(B, M * K_NB, 128)
    crep = jnp.broadcast_to(centers[:, :, None, :],
                            (B, M, K_NB, 3)).reshape(B, M * K_NB, 3)
    out = _mlp_pool(rows, crep, valid, Cf, params)
    return out, centers


def kernel(x, W1_1, b1_1, W1_2, b1_2, W1_3, b1_3,
           W2_1, b2_1, W2_2, b2_2, W2_3, b2_3):
    B, N, _ = x.shape
    feat = x[:, :, 3:]
    pos = x[:, :, :3]
    params1 = [(W1_1, b1_1), (W1_2, b1_2), (W1_3, b1_3)]
    params2 = [(W2_1, b2_1), (W2_2, b2_2), (W2_3, b2_3)]
    f1, p1 = _set_conv(feat, pos, 0.5, N // 2, params1)
    f2, p2 = _set_conv(f1, p1, 1.0, N // 8, params2)
    M2 = f2.shape[1]
    batch = jnp.repeat(jnp.arange(B, dtype=jnp.int32), M2)
    return (f2.reshape(B * M2, -1), p2.reshape(B * M2, 3), batch)
